# baseline (speedup 1.0000x reference)
import jax
import jax.numpy as jnp
from jax import lax
from jax.experimental import pallas as pl
from jax.experimental.pallas import tpu as pltpu
from jax.experimental.pallas import tpu_sc as plsc

NB = 32
NBINS = NB * NB
LANES = 16
NWORKERS = 32
TOTAL = 4 * 2048 * 2048
PER_WORKER = TOTAL // NWORKERS
CHUNK = 16384
NCHUNKS = PER_WORKER // CHUNK
VECS = CHUNK // LANES
UNROLL = 16
HIST_WORDS = LANES * NBINS

_MAGIC = 12582912.0
_MAGIC_BITS = 0x4B400000
_BIAS = ((-((_MAGIC_BITS << 5) + _MAGIC_BITS)) % (1 << 32))


ROWS_PER_WORKER = PER_WORKER // 2048
ROWS_PER_CHUNK = CHUNK // 2048
WORKERS_PER_IMAGE = (2048 * 2048) // PER_WORKER


def _sc_hist_kernel(i_hbm, j_hbm, out_hbm,
                    ib0, jb0, ib1, jb1, hist, hist2, sem0, sem1):
    wid = lax.axis_index("s") * 2 + lax.axis_index("c")
    img = wid // WORKERS_PER_IMAGE
    row0 = (wid % WORKERS_PER_IMAGE) * ROWS_PER_WORKER

    ones = jnp.ones((LANES,), jnp.float32)
    zeros = jnp.zeros((LANES,), jnp.float32)
    scale = jnp.float32(31.0 / 255.0)
    mlane = (lax.iota(jnp.int32, LANES) * NBINS).astype(jnp.float32) + jnp.float32(_MAGIC)
    bias = jnp.int32(_BIAS)

    @plsc.parallel_loop(0, HIST_WORDS // LANES, unroll=4)
    def zero_body(i):
        hist[pl.ds(i * LANES, LANES)] = zeros
        hist2[pl.ds(i * LANES, LANES)] = zeros

    bufs = ((ib0, jb0, sem0), (ib1, jb1, sem1))
    VECS_PER_ROW = 2048 // LANES

    def issue(b, c):
        ib, jb, sem = bufs[b]
        r = row0 + c * ROWS_PER_CHUNK
        pltpu.async_copy(i_hbm.at[img, 0, pl.ds(r, ROWS_PER_CHUNK), :], ib, sem)
        pltpu.async_copy(j_hbm.at[img, 0, pl.ds(r, ROWS_PER_CHUNK), :], jb, sem)

    issue(0, 0)
    issue(1, 1)

    def chunk_pair(cc, carry):
        for b in range(2):
            c = cc * 2 + b
            ib, jb, sem = bufs[b]
            pltpu.make_async_copy(
                i_hbm.at[img, 0, pl.ds(row0, ROWS_PER_CHUNK), :], ib, sem).wait()
            pltpu.make_async_copy(
                j_hbm.at[img, 0, pl.ds(row0, ROWS_PER_CHUNK), :], jb, sem).wait()

            @plsc.parallel_loop(0, VECS, step=2, unroll=8)
            def vec_body(kk):
                r = kk // VECS_PER_ROW
                col = (kk % VECS_PER_ROW) * LANES
                for t, hb in ((0, hist), (1, hist2)):
                    iv = ib[r, pl.ds(col + t * LANES, LANES)]
                    jv = jb[r, pl.ds(col + t * LANES, LANES)]
                    bi = plsc.bitcast(iv * scale + jnp.float32(_MAGIC), jnp.int32)
                    bj = plsc.bitcast(jv * scale + mlane, jnp.int32)
                    idx = (bi << 5) + bj + bias
                    plsc.addupdate_scatter(hb, [idx], ones)

            @pl.when(c + 2 < NCHUNKS)
            def _():
                issue(b, c + 2)
        return carry
    lax.fori_loop(0, NCHUNKS // 2, chunk_pair, 0)

    @plsc.parallel_loop(0, HIST_WORDS // LANES, unroll=4)
    def merge_body(i):
        o = i * LANES
        hist[pl.ds(o, LANES)] = hist[pl.ds(o, LANES)] + hist2[pl.ds(o, LANES)]

    pltpu.sync_copy(hist, out_hbm.at[pl.ds(wid * HIST_WORDS, HIST_WORDS)])


def _mi_kernel(x_ref, o_ref):
    x = x_ref[...]
    h = jnp.sum(x, axis=0, keepdims=True)
    total = jnp.sum(h)
    p = h / total

    col = lax.broadcasted_iota(jnp.int32, (NB, NBINS), 1)
    row = lax.broadcasted_iota(jnp.int32, (NB, NBINS), 0)
    mask_i = (col // NB) == row
    mask_j = (col % NB) == row

    pb = jnp.broadcast_to(p, (NB, NBINS))
    zero = jnp.zeros((), jnp.float32)
    i_prob = jnp.sum(jnp.where(mask_i, pb, zero), axis=1, keepdims=True)
    j_prob = jnp.sum(jnp.where(mask_j, pb, zero), axis=1, keepdims=True)
    log_i = jnp.log(i_prob + 1e-5)
    log_j = jnp.log(j_prob + 1e-5)
    log_i_k = jnp.sum(jnp.where(mask_i, jnp.broadcast_to(log_i, (NB, NBINS)), zero),
                      axis=0, keepdims=True)
    log_j_k = jnp.sum(jnp.where(mask_j, jnp.broadcast_to(log_j, (NB, NBINS)), zero),
                      axis=0, keepdims=True)
    mi = jnp.sum(p * (jnp.log(p + 1e-5) - log_i_k - log_j_k))
    o_ref[...] = jnp.broadcast_to(1.0 / (1.0 + jnp.exp(mi)), (1, 1))


@jax.jit
def kernel(I, J):
    mesh = plsc.VectorSubcoreMesh(core_axis_name="c", subcore_axis_name="s")
    parts = pl.kernel(
        _sc_hist_kernel,
        jax.ShapeDtypeStruct((NWORKERS * HIST_WORDS,), jnp.float32),
        mesh=mesh,
        scratch_types=[
            pltpu.VMEM((ROWS_PER_CHUNK, 2048), jnp.float32),
            pltpu.VMEM((ROWS_PER_CHUNK, 2048), jnp.float32),
            pltpu.VMEM((ROWS_PER_CHUNK, 2048), jnp.float32),
            pltpu.VMEM((ROWS_PER_CHUNK, 2048), jnp.float32),
            pltpu.VMEM((HIST_WORDS,), jnp.float32),
            pltpu.VMEM((HIST_WORDS,), jnp.float32),
            pltpu.SemaphoreType.DMA,
            pltpu.SemaphoreType.DMA,
        ],
        compiler_params=pltpu.CompilerParams(needs_layout_passes=False),
    )(I, J)

    x = parts.reshape(NWORKERS * LANES, NBINS)
    out = pl.pallas_call(
        _mi_kernel,
        out_shape=jax.ShapeDtypeStruct((1, 1), jnp.float32),
    )(x)
    return out.reshape(())


# speedup vs baseline: 1.0030x; 1.0030x over previous
import jax
import jax.numpy as jnp
from jax import lax
from jax.experimental import pallas as pl
from jax.experimental.pallas import tpu as pltpu
from jax.experimental.pallas import tpu_sc as plsc

NB = 32
NBINS = NB * NB
LANES = 16
NWORKERS = 32
TOTAL = 4 * 2048 * 2048
PER_WORKER = TOTAL // NWORKERS
CHUNK = 16384
NCHUNKS = PER_WORKER // CHUNK
VECS = CHUNK // LANES
UNROLL = 16
HIST_WORDS = LANES * NBINS

_MAGIC = 12582912.0
_MAGIC_BITS = 0x4B400000
_BIAS = ((-((_MAGIC_BITS << 5) + _MAGIC_BITS)) % (1 << 32))


ROWS_PER_WORKER = PER_WORKER // 2048
ROWS_PER_CHUNK = CHUNK // 2048
WORKERS_PER_IMAGE = (2048 * 2048) // PER_WORKER


def _sc_hist_kernel(i_hbm, j_hbm, out_hbm,
                    ib0, jb0, ib1, jb1, hist, hist2, sem0, sem1):
    wid = lax.axis_index("s") * 2 + lax.axis_index("c")
    img = wid // WORKERS_PER_IMAGE
    row0 = (wid % WORKERS_PER_IMAGE) * ROWS_PER_WORKER

    ones = jnp.ones((LANES,), jnp.float32)
    zeros = jnp.zeros((LANES,), jnp.float32)
    scale = jnp.float32(31.0 / 255.0)
    mlane = (lax.iota(jnp.int32, LANES) * NBINS).astype(jnp.float32) + jnp.float32(_MAGIC)
    bias = jnp.int32(_BIAS)

    @plsc.parallel_loop(0, HIST_WORDS // LANES, unroll=4)
    def zero_body(i):
        hist[pl.ds(i * LANES, LANES)] = zeros
        hist2[pl.ds(i * LANES, LANES)] = zeros

    bufs = ((ib0, jb0, sem0), (ib1, jb1, sem1))
    VECS_PER_ROW = 2048 // LANES

    def issue(b, c):
        ib, jb, sem = bufs[b]
        r = row0 + c * ROWS_PER_CHUNK
        pltpu.async_copy(i_hbm.at[img, 0, pl.ds(r, ROWS_PER_CHUNK), :], ib, sem)
        pltpu.async_copy(j_hbm.at[img, 0, pl.ds(r, ROWS_PER_CHUNK), :], jb, sem)

    issue(0, 0)
    issue(1, 1)

    def chunk_pair(cc, carry):
        for b in range(2):
            c = cc * 2 + b
            ib, jb, sem = bufs[b]
            pltpu.make_async_copy(
                i_hbm.at[img, 0, pl.ds(row0, ROWS_PER_CHUNK), :], ib, sem).wait()
            pltpu.make_async_copy(
                j_hbm.at[img, 0, pl.ds(row0, ROWS_PER_CHUNK), :], jb, sem).wait()

            @plsc.parallel_loop(0, VECS, step=2, unroll=4)
            def vec_body(kk):
                r = kk // VECS_PER_ROW
                col = (kk % VECS_PER_ROW) * LANES
                for t, hb in ((0, hist), (1, hist2)):
                    iv = ib[r, pl.ds(col + t * LANES, LANES)]
                    jv = jb[r, pl.ds(col + t * LANES, LANES)]
                    bi = plsc.bitcast(iv * scale + jnp.float32(_MAGIC), jnp.int32)
                    bj = plsc.bitcast(jv * scale + mlane, jnp.int32)
                    idx = (bi << 5) + bj + bias
                    plsc.addupdate_scatter(hb, [idx], ones)

            @pl.when(c + 2 < NCHUNKS)
            def _():
                issue(b, c + 2)
        return carry
    lax.fori_loop(0, NCHUNKS // 2, chunk_pair, 0)

    @plsc.parallel_loop(0, HIST_WORDS // LANES, unroll=4)
    def merge_body(i):
        o = i * LANES
        hist[pl.ds(o, LANES)] = hist[pl.ds(o, LANES)] + hist2[pl.ds(o, LANES)]

    pltpu.sync_copy(hist, out_hbm.at[pl.ds(wid * HIST_WORDS, HIST_WORDS)])


def _mi_kernel(x_ref, o_ref):
    x = x_ref[...]
    h = jnp.sum(x, axis=0, keepdims=True)
    total = jnp.sum(h)
    p = h / total

    col = lax.broadcasted_iota(jnp.int32, (NB, NBINS), 1)
    row = lax.broadcasted_iota(jnp.int32, (NB, NBINS), 0)
    mask_i = (col // NB) == row
    mask_j = (col % NB) == row

    pb = jnp.broadcast_to(p, (NB, NBINS))
    zero = jnp.zeros((), jnp.float32)
    i_prob = jnp.sum(jnp.where(mask_i, pb, zero), axis=1, keepdims=True)
    j_prob = jnp.sum(jnp.where(mask_j, pb, zero), axis=1, keepdims=True)
    log_i = jnp.log(i_prob + 1e-5)
    log_j = jnp.log(j_prob + 1e-5)
    log_i_k = jnp.sum(jnp.where(mask_i, jnp.broadcast_to(log_i, (NB, NBINS)), zero),
                      axis=0, keepdims=True)
    log_j_k = jnp.sum(jnp.where(mask_j, jnp.broadcast_to(log_j, (NB, NBINS)), zero),
                      axis=0, keepdims=True)
    mi = jnp.sum(p * (jnp.log(p + 1e-5) - log_i_k - log_j_k))
    o_ref[...] = jnp.broadcast_to(1.0 / (1.0 + jnp.exp(mi)), (1, 1))


@jax.jit
def kernel(I, J):
    mesh = plsc.VectorSubcoreMesh(core_axis_name="c", subcore_axis_name="s")
    parts = pl.kernel(
        _sc_hist_kernel,
        jax.ShapeDtypeStruct((NWORKERS * HIST_WORDS,), jnp.float32),
        mesh=mesh,
        scratch_types=[
            pltpu.VMEM((ROWS_PER_CHUNK, 2048), jnp.float32),
            pltpu.VMEM((ROWS_PER_CHUNK, 2048), jnp.float32),
            pltpu.VMEM((ROWS_PER_CHUNK, 2048), jnp.float32),
            pltpu.VMEM((ROWS_PER_CHUNK, 2048), jnp.float32),
            pltpu.VMEM((HIST_WORDS,), jnp.float32),
            pltpu.VMEM((HIST_WORDS,), jnp.float32),
            pltpu.SemaphoreType.DMA,
            pltpu.SemaphoreType.DMA,
        ],
        compiler_params=pltpu.CompilerParams(needs_layout_passes=False),
    )(I, J)

    x = parts.reshape(NWORKERS * LANES, NBINS)
    out = pl.pallas_call(
        _mi_kernel,
        out_shape=jax.ShapeDtypeStruct((1, 1), jnp.float32),
    )(x)
    return out.reshape(())
